# Initial kernel scaffold; baseline (speedup 1.0000x reference)
import jax
import jax.numpy as jnp
from jax.experimental import pallas as pl


def kernel(species, edge_src, edge_dst, distances, switch, W_sp, aw1_W, aw1_b, f0_W, f0_b, f1_W, f1_b, f2_W, f2_b, aw2_W, aw2_b, aw3_W, aw3_b):
    raise NotImplementedError("write your pallas kernel here")


# speedup vs baseline: 2.1193x; 2.1193x over previous
import functools

import jax
import jax.numpy as jnp
import numpy as np
from jax import lax
from jax.experimental import pallas as pl
from jax.experimental.pallas import tpu as pltpu
from jax.experimental.pallas import tpu_sc as plsc

N = 50000
E = 800000
DIM = 64
NBASIS = 16
NSPEC = 94
NLAYERS = 3
CUTOFF = 5.0
LOG2 = float(np.log(2.0))

BN = 2000
EB = 4096

NC = 2
NS = 16
HD = DIM // NC
CK = 256
G = 128
GR = CK // G
EPAD = 16 * 196 * CK
EPT = EPAD // NS
NCHUNK = EPT // CK
NPAD = 51200
RPT = NPAD // NS
ZR = 200


def _ssp(x):
    return jax.nn.softplus(x) - LOG2


def _embed_body(sp_ref, wsp_ref, out_ref):
    sp = sp_ref[0, 0, :]
    oh = (sp[:, None] == lax.broadcasted_iota(jnp.int32, (BN, 128), 1))
    out_ref[...] = jnp.dot(oh.astype(jnp.float32), wsp_ref[...],
                           preferred_element_type=jnp.float32)


def _embed(species, wsp_pad):
    return pl.pallas_call(
        _embed_body,
        grid=(N // BN,),
        in_specs=[
            pl.BlockSpec((1, 1, BN), lambda i: (i, 0, 0)),
            pl.BlockSpec((128, DIM), lambda i: (0, 0)),
        ],
        out_specs=pl.BlockSpec((BN, DIM), lambda i: (i, 0)),
        out_shape=jax.ShapeDtypeStruct((N, DIM), jnp.float32),
    )(species.reshape(N // BN, 1, BN), wsp_pad)


def _filter_body(d_ref, sw_ref, f0w_ref, f0b_ref, f1w_ref, f1b_ref,
                 f2w_ref, f2b_ref, o0_ref, o1_ref, o2_ref):
    d = d_ref[0, 0, :]
    sw = sw_ref[0, 0, :]
    delta = CUTOFF / NBASIS
    gamma = 1.0 / (2.0 * delta * delta)
    mus = lax.broadcasted_iota(jnp.int32, (EB, NBASIS), 1).astype(jnp.float32) \
        * (CUTOFF / (NBASIS - 1))
    rb = jnp.exp(-gamma * (d[:, None] - mus) ** 2)
    outs = (o0_ref, o1_ref, o2_ref)
    for l in range(NLAYERS):
        h = _ssp(jnp.dot(rb, f0w_ref[l], preferred_element_type=jnp.float32)
                 + f0b_ref[l][None, :])
        h = _ssp(jnp.dot(h, f1w_ref[l], preferred_element_type=jnp.float32)
                 + f1b_ref[l][None, :])
        w = jnp.dot(h, f2w_ref[l], preferred_element_type=jnp.float32) \
            + f2b_ref[l][None, :]
        wsp = _ssp(w) * sw[:, None]
        outs[l][0, :, :] = wsp[:, :HD]
        outs[l][1, :, :] = wsp[:, HD:]


def _filter(distances, switch, f0_W, f0_b, f1_W, f1_b, f2_W, f2_b):
    wspec = lambda s: pl.BlockSpec(s, lambda i: tuple(0 for _ in s))
    oshape = jax.ShapeDtypeStruct((NC, EPAD, HD), jnp.float32)
    return pl.pallas_call(
        _filter_body,
        grid=(EPAD // EB,),
        in_specs=[
            pl.BlockSpec((1, 1, EB), lambda i: (i, 0, 0)),
            pl.BlockSpec((1, 1, EB), lambda i: (i, 0, 0)),
            wspec((NLAYERS, NBASIS, 64)), wspec((NLAYERS, 64)),
            wspec((NLAYERS, 64, 64)), wspec((NLAYERS, 64)),
            wspec((NLAYERS, 64, DIM)), wspec((NLAYERS, DIM)),
        ],
        out_specs=[pl.BlockSpec((NC, EB, HD), lambda i: (0, i, 0))] * NLAYERS,
        out_shape=[oshape] * NLAYERS,
    )(distances.reshape(EPAD // EB, 1, EB), switch.reshape(EPAD // EB, 1, EB),
      f0_W, f0_b, f1_W, f1_b, f2_W, f2_b)


def _node_in_body(x_ref, w_ref, b_ref, out_ref):
    y = jnp.dot(x_ref[...], w_ref[...], preferred_element_type=jnp.float32) \
        + b_ref[0, :][None, :]
    out_ref[0, :, :] = y[:, :HD]
    out_ref[1, :, :] = y[:, HD:]


def _node_in(xi, w, b):
    return pl.pallas_call(
        _node_in_body,
        grid=(N // BN,),
        in_specs=[
            pl.BlockSpec((BN, DIM), lambda i: (i, 0)),
            pl.BlockSpec((DIM, DIM), lambda i: (0, 0)),
            pl.BlockSpec((1, DIM), lambda i: (0, 0)),
        ],
        out_specs=pl.BlockSpec((NC, BN, HD), lambda i: (0, i, 0)),
        out_shape=jax.ShapeDtypeStruct((NC, N, HD), jnp.float32),
    )(xi, w, b.reshape(1, DIM))


def _node_out_body(acc_ref, xp_ref, w2_ref, b2_ref, w3_ref, b3_ref, out_ref):
    t = jnp.concatenate([acc_ref[0, :, :], acc_ref[1, :, :]], axis=-1)
    u = _ssp(jnp.dot(t, w2_ref[...], preferred_element_type=jnp.float32)
             + b2_ref[0, :][None, :])
    out_ref[...] = jnp.dot(u, w3_ref[...], preferred_element_type=jnp.float32) \
        + b3_ref[0, :][None, :] + xp_ref[...]


def _node_out(acc, xi_prev, w2, b2, w3, b3):
    return pl.pallas_call(
        _node_out_body,
        grid=(N // BN,),
        in_specs=[
            pl.BlockSpec((NC, BN, HD), lambda i: (0, i, 0)),
            pl.BlockSpec((BN, DIM), lambda i: (i, 0)),
            pl.BlockSpec((DIM, DIM), lambda i: (0, 0)),
            pl.BlockSpec((1, DIM), lambda i: (0, 0)),
            pl.BlockSpec((DIM, DIM), lambda i: (0, 0)),
            pl.BlockSpec((1, DIM), lambda i: (0, 0)),
        ],
        out_specs=pl.BlockSpec((BN, DIM), lambda i: (i, 0)),
        out_shape=jax.ShapeDtypeStruct((N, DIM), jnp.float32),
    )(acc, xi_prev, w2, b2.reshape(1, DIM), w3, b3.reshape(1, DIM))


def _sc_conv_body(xi2_hbm, wp_hbm, srcr_hbm, dstr_hbm, out_hbm,
                  sidx, didx, g, w, acc, sem):
    c = lax.axis_index("c")
    s = lax.axis_index("s")

    zv = jnp.zeros((16,), jnp.float32)

    def zb_body(r, _):
        g[r, pl.ds(0, 16)] = zv
        g[r, pl.ds(16, 16)] = zv
        return _
    lax.fori_loop(0, ZR, zb_body, 0, unroll=8)
    for q in range(RPT // ZR):
        z0 = pl.multiple_of(s * RPT + q * ZR, ZR)
        pltpu.sync_copy(g.at[pl.ds(0, ZR)], acc.at[pl.ds(z0, ZR)])
    plsc.subcore_barrier()

    def chunk(i, _):
        base = pl.multiple_of(s * EPT + i * CK, CK)
        rbase = pl.multiple_of(base // G, GR)
        pltpu.sync_copy(srcr_hbm.at[pl.ds(rbase, GR)], sidx)
        pltpu.sync_copy(dstr_hbm.at[c, pl.ds(rbase, GR)], didx)
        cps = [pltpu.async_copy(xi2_hbm.at[didx.at[j]],
                                g.at[pl.ds(j * G, G)], sem)
               for j in range(GR)]
        pltpu.sync_copy(wp_hbm.at[c, pl.ds(base, CK)], w)
        for cp in cps:
            cp.wait()

        def mul(r0, _):
            for u in range(8):
                r = r0 * 8 + u
                g[r, pl.ds(0, 16)] = g[r, pl.ds(0, 16)] * w[r, pl.ds(0, 16)]
                g[r, pl.ds(16, 16)] = g[r, pl.ds(16, 16)] * w[r, pl.ds(16, 16)]
            return _
        lax.fori_loop(0, CK // 8, mul, 0)

        for j in range(GR):
            pltpu.sync_copy(g.at[pl.ds(j * G, G)], acc.at[sidx.at[j]], add=True)
        return _

    lax.fori_loop(0, NCHUNK, chunk, 0)
    plsc.subcore_barrier()
    for q in range(RPT // ZR):
        r0 = pl.multiple_of(s * RPT + q * ZR, ZR)
        pltpu.sync_copy(acc.at[pl.ds(r0, ZR)], out_hbm.at[c, pl.ds(r0, ZR)])


def _sc_conv(xi_h, wp, srcr, dstr):
    mesh = plsc.VectorSubcoreMesh(core_axis_name="c", subcore_axis_name="s")
    return pl.kernel(
        _sc_conv_body,
        out_type=jax.ShapeDtypeStruct((NC, NPAD, HD), jnp.float32),
        mesh=mesh,
        compiler_params=pltpu.CompilerParams(use_tc_tiling_on_sc=False),
        scratch_types=[
            pltpu.VMEM((GR, G), jnp.int32),
            pltpu.VMEM((GR, G), jnp.int32),
            pltpu.VMEM((CK, HD), jnp.float32),
            pltpu.VMEM((CK, HD), jnp.float32),
            pltpu.VMEM_SHARED((NPAD, HD), jnp.float32),
            pltpu.SemaphoreType.DMA,
        ],
    )(xi_h.reshape(NC * N, HD), wp, srcr, dstr)


def kernel(species, edge_src, edge_dst, distances, switch,
           W_sp, aw1_W, aw1_b, f0_W, f0_b, f1_W, f1_b, f2_W, f2_b,
           aw2_W, aw2_b, aw3_W, aw3_b):
    species = species.astype(jnp.int32)
    edge_src = edge_src.astype(jnp.int32)
    edge_dst = edge_dst.astype(jnp.int32)

    wsp_pad = jnp.zeros((128, DIM), jnp.float32).at[:NSPEC].set(W_sp)
    xi_prev = _embed(species, wsp_pad)

    dpad = jnp.pad(distances, (0, EPAD - E))
    swpad = jnp.pad(switch, (0, EPAD - E))
    wps = _filter(dpad, swpad, f0_W, f0_b, f1_W, f1_b, f2_W, f2_b)

    srcr = jnp.pad(edge_src, (0, EPAD - E)).reshape(EPAD // G, G)
    dstr = jnp.pad(jnp.stack([edge_dst, edge_dst + N]),
                   ((0, 0), (0, EPAD - E))).reshape(NC, EPAD // G, G)

    for l in range(NLAYERS):
        xi_h = _node_in(xi_prev, aw1_W[l], aw1_b[l])
        acc = _sc_conv(xi_h, wps[l], srcr, dstr)
        xi_prev = _node_out(acc, xi_prev, aw2_W[l], aw2_b[l], aw3_W[l], aw3_b[l])
    return xi_prev


# speedup vs baseline: 2.5581x; 1.2070x over previous
import functools

import jax
import jax.numpy as jnp
import numpy as np
from jax import lax
from jax.experimental import pallas as pl
from jax.experimental.pallas import tpu as pltpu
from jax.experimental.pallas import tpu_sc as plsc

N = 50000
E = 800000
DIM = 64
NBASIS = 16
NSPEC = 94
NLAYERS = 3
CUTOFF = 5.0
LOG2 = float(np.log(2.0))

BN = 2000
EB = 4096

NC = 2
NS = 16
HD = DIM // NC
CK = 256
G = 128
GR = CK // G
EPAD = 16 * 196 * CK
EPT = EPAD // NS
NCHUNK = EPT // CK
NPAD = 51200
RPT = NPAD // NS
ZR = 200


def _ssp(x):
    t = jnp.minimum(x * np.float32(1.4426950408889634), np.float32(126.0))
    return np.float32(LOG2) * (jnp.log2(1.0 + jnp.exp2(t)) - 1.0)


def _embed_body(sp_ref, wsp_ref, out_ref):
    sp = sp_ref[0, 0, :]
    oh = (sp[:, None] == lax.broadcasted_iota(jnp.int32, (BN, 128), 1))
    out_ref[...] = jnp.dot(oh.astype(jnp.float32), wsp_ref[...],
                           preferred_element_type=jnp.float32)


def _embed(species, wsp_pad):
    return pl.pallas_call(
        _embed_body,
        grid=(N // BN,),
        in_specs=[
            pl.BlockSpec((1, 1, BN), lambda i: (i, 0, 0)),
            pl.BlockSpec((128, DIM), lambda i: (0, 0)),
        ],
        out_specs=pl.BlockSpec((BN, DIM), lambda i: (i, 0)),
        out_shape=jax.ShapeDtypeStruct((N, DIM), jnp.float32),
    )(species.reshape(N // BN, 1, BN), wsp_pad)


def _filter_body(d_ref, sw_ref, f0w_ref, f0b_ref, f1w_ref, f1b_ref,
                 f2w_ref, f2b_ref, o0_ref, o1_ref, o2_ref):
    d = d_ref[0, 0, :]
    sw = sw_ref[0, 0, :]
    delta = CUTOFF / NBASIS
    gamma = 1.0 / (2.0 * delta * delta)
    mus = lax.broadcasted_iota(jnp.int32, (EB, NBASIS), 1).astype(jnp.float32) \
        * (CUTOFF / (NBASIS - 1))
    rb = jnp.exp(-gamma * (d[:, None] - mus) ** 2)
    outs = (o0_ref, o1_ref, o2_ref)
    for l in range(NLAYERS):
        h = _ssp(jnp.dot(rb, f0w_ref[l], preferred_element_type=jnp.float32)
                 + f0b_ref[l][None, :])
        h = _ssp(jnp.dot(h, f1w_ref[l], preferred_element_type=jnp.float32)
                 + f1b_ref[l][None, :])
        w = jnp.dot(h, f2w_ref[l], preferred_element_type=jnp.float32) \
            + f2b_ref[l][None, :]
        wsp = _ssp(w) * sw[:, None]
        outs[l][0, :, :] = wsp[:, :HD]
        outs[l][1, :, :] = wsp[:, HD:]


def _filter(distances, switch, f0_W, f0_b, f1_W, f1_b, f2_W, f2_b):
    wspec = lambda s: pl.BlockSpec(s, lambda i: tuple(0 for _ in s))
    oshape = jax.ShapeDtypeStruct((NC, EPAD, HD), jnp.float32)
    return pl.pallas_call(
        _filter_body,
        grid=(EPAD // EB,),
        in_specs=[
            pl.BlockSpec((1, 1, EB), lambda i: (i, 0, 0)),
            pl.BlockSpec((1, 1, EB), lambda i: (i, 0, 0)),
            wspec((NLAYERS, NBASIS, 64)), wspec((NLAYERS, 64)),
            wspec((NLAYERS, 64, 64)), wspec((NLAYERS, 64)),
            wspec((NLAYERS, 64, DIM)), wspec((NLAYERS, DIM)),
        ],
        out_specs=[pl.BlockSpec((NC, EB, HD), lambda i: (0, i, 0))] * NLAYERS,
        out_shape=[oshape] * NLAYERS,
    )(distances.reshape(EPAD // EB, 1, EB), switch.reshape(EPAD // EB, 1, EB),
      f0_W, f0_b, f1_W, f1_b, f2_W, f2_b)


def _node_in_body(x_ref, w_ref, b_ref, out_ref):
    c = pl.program_id(1)
    y = jnp.dot(x_ref[...], w_ref[...], preferred_element_type=jnp.float32) \
        + b_ref[0, :][None, :]
    out_ref[...] = jnp.where(c == 0, y[:, :HD], y[:, HD:])


def _node_in(xi, w, b):
    return pl.pallas_call(
        _node_in_body,
        grid=(N // BN, NC),
        in_specs=[
            pl.BlockSpec((BN, DIM), lambda i, c: (i, 0)),
            pl.BlockSpec((DIM, DIM), lambda i, c: (0, 0)),
            pl.BlockSpec((1, DIM), lambda i, c: (0, 0)),
        ],
        out_specs=pl.BlockSpec((BN, HD), lambda i, c: (c * (N // BN) + i, 0)),
        out_shape=jax.ShapeDtypeStruct((NC * N, HD), jnp.float32),
    )(xi, w, b.reshape(1, DIM))


def _node_out_body(acc_ref, xp_ref, w2_ref, b2_ref, w3_ref, b3_ref, out_ref):
    t = jnp.concatenate([acc_ref[0, :, :], acc_ref[1, :, :]], axis=-1)
    u = _ssp(jnp.dot(t, w2_ref[...], preferred_element_type=jnp.float32)
             + b2_ref[0, :][None, :])
    out_ref[...] = jnp.dot(u, w3_ref[...], preferred_element_type=jnp.float32) \
        + b3_ref[0, :][None, :] + xp_ref[...]


def _node_out(acc, xi_prev, w2, b2, w3, b3):
    return pl.pallas_call(
        _node_out_body,
        grid=(N // BN,),
        in_specs=[
            pl.BlockSpec((NC, BN, HD), lambda i: (0, i, 0)),
            pl.BlockSpec((BN, DIM), lambda i: (i, 0)),
            pl.BlockSpec((DIM, DIM), lambda i: (0, 0)),
            pl.BlockSpec((1, DIM), lambda i: (0, 0)),
            pl.BlockSpec((DIM, DIM), lambda i: (0, 0)),
            pl.BlockSpec((1, DIM), lambda i: (0, 0)),
        ],
        out_specs=pl.BlockSpec((BN, DIM), lambda i: (i, 0)),
        out_shape=jax.ShapeDtypeStruct((N, DIM), jnp.float32),
    )(acc, xi_prev, w2, b2.reshape(1, DIM), w3, b3.reshape(1, DIM))


def _sc_conv_body(xi2_hbm, wp_hbm, srcr_hbm, dstr_hbm, out_hbm,
                  sidx, didx, g, w, acc, sem):
    c = lax.axis_index("c")
    s = lax.axis_index("s")

    zv = jnp.zeros((16,), jnp.float32)

    def zb_body(r, _):
        g[r, pl.ds(0, 16)] = zv
        g[r, pl.ds(16, 16)] = zv
        return _
    lax.fori_loop(0, ZR, zb_body, 0, unroll=8)
    for q in range(RPT // ZR):
        z0 = pl.multiple_of(s * RPT + q * ZR, ZR)
        pltpu.sync_copy(g.at[pl.ds(0, ZR)], acc.at[pl.ds(z0, ZR)])
    plsc.subcore_barrier()

    def chunk(i, _):
        base = pl.multiple_of(s * EPT + i * CK, CK)
        rbase = pl.multiple_of(base // G, GR)
        pltpu.sync_copy(srcr_hbm.at[pl.ds(rbase, GR)], sidx)
        pltpu.sync_copy(dstr_hbm.at[c, pl.ds(rbase, GR)], didx)
        cps = [pltpu.async_copy(xi2_hbm.at[didx.at[j]],
                                g.at[pl.ds(j * G, G)], sem)
               for j in range(GR)]
        pltpu.sync_copy(wp_hbm.at[c, pl.ds(base, CK)], w)
        for cp in cps:
            cp.wait()

        def mul(r0, _):
            for u in range(8):
                r = r0 * 8 + u
                g[r, pl.ds(0, 16)] = g[r, pl.ds(0, 16)] * w[r, pl.ds(0, 16)]
                g[r, pl.ds(16, 16)] = g[r, pl.ds(16, 16)] * w[r, pl.ds(16, 16)]
            return _
        lax.fori_loop(0, CK // 8, mul, 0)

        for j in range(GR):
            pltpu.sync_copy(g.at[pl.ds(j * G, G)], acc.at[sidx.at[j]], add=True)
        return _

    lax.fori_loop(0, NCHUNK, chunk, 0)
    plsc.subcore_barrier()
    for q in range(RPT // ZR):
        r0 = pl.multiple_of(s * RPT + q * ZR, ZR)
        pltpu.sync_copy(acc.at[pl.ds(r0, ZR)], out_hbm.at[c, pl.ds(r0, ZR)])


def _sc_conv(xi_h, wp, srcr, dstr):
    mesh = plsc.VectorSubcoreMesh(core_axis_name="c", subcore_axis_name="s")
    return pl.kernel(
        _sc_conv_body,
        out_type=jax.ShapeDtypeStruct((NC, NPAD, HD), jnp.float32),
        mesh=mesh,
        compiler_params=pltpu.CompilerParams(use_tc_tiling_on_sc=False),
        scratch_types=[
            pltpu.VMEM((GR, G), jnp.int32),
            pltpu.VMEM((GR, G), jnp.int32),
            pltpu.VMEM((CK, HD), jnp.float32),
            pltpu.VMEM((CK, HD), jnp.float32),
            pltpu.VMEM_SHARED((NPAD, HD), jnp.float32),
            pltpu.SemaphoreType.DMA,
        ],
    )(xi_h, wp, srcr, dstr)


def kernel(species, edge_src, edge_dst, distances, switch,
           W_sp, aw1_W, aw1_b, f0_W, f0_b, f1_W, f1_b, f2_W, f2_b,
           aw2_W, aw2_b, aw3_W, aw3_b):
    species = species.astype(jnp.int32)
    edge_src = edge_src.astype(jnp.int32)
    edge_dst = edge_dst.astype(jnp.int32)

    wsp_pad = jnp.zeros((128, DIM), jnp.float32).at[:NSPEC].set(W_sp)
    xi_prev = _embed(species, wsp_pad)

    dpad = jnp.pad(distances, (0, EPAD - E))
    swpad = jnp.pad(switch, (0, EPAD - E))
    wps = _filter(dpad, swpad, f0_W, f0_b, f1_W, f1_b, f2_W, f2_b)

    srcr = jnp.pad(edge_src, (0, EPAD - E)).reshape(EPAD // G, G)
    dstr = jnp.pad(jnp.stack([edge_dst, edge_dst + N]),
                   ((0, 0), (0, EPAD - E))).reshape(NC, EPAD // G, G)

    for l in range(NLAYERS):
        xi_h = _node_in(xi_prev, aw1_W[l], aw1_b[l])
        acc = _sc_conv(xi_h, wps[l], srcr, dstr)
        xi_prev = _node_out(acc, xi_prev, aw2_W[l], aw2_b[l], aw3_W[l], aw3_b[l])
    return xi_prev


# speedup vs baseline: 2.8236x; 1.1038x over previous
import functools

import jax
import jax.numpy as jnp
import numpy as np
from jax import lax
from jax.experimental import pallas as pl
from jax.experimental.pallas import tpu as pltpu
from jax.experimental.pallas import tpu_sc as plsc

N = 50000
E = 800000
DIM = 64
NBASIS = 16
NSPEC = 94
NLAYERS = 3
CUTOFF = 5.0
LOG2 = float(np.log(2.0))

BN = 2000
EB = 4096
EB2 = EB // 2

NC = 2
NS = 16
HD = DIM // NC
CK = 256
G = 128
GR = CK // G
EPAD = 16 * 196 * CK
EPT = EPAD // NS
NCHUNK = EPT // CK
NPAD = 51200
RPT = NPAD // NS
ZR = 200


def _ssp(x):
    t = jnp.minimum(x * np.float32(1.4426950408889634), np.float32(126.0))
    return np.float32(LOG2) * (jnp.log2(1.0 + jnp.exp2(t)) - 1.0)


def _embed_body(sp_ref, wsp_ref, out_ref):
    sp = sp_ref[0, 0, :]
    oh = (sp[:, None] == lax.broadcasted_iota(jnp.int32, (BN, 128), 1))
    out_ref[...] = jnp.dot(oh.astype(jnp.float32), wsp_ref[...],
                           preferred_element_type=jnp.float32)


def _embed(species, wsp_pad):
    return pl.pallas_call(
        _embed_body,
        grid=(N // BN,),
        in_specs=[
            pl.BlockSpec((1, 1, BN), lambda i: (i, 0, 0)),
            pl.BlockSpec((128, DIM), lambda i: (0, 0)),
        ],
        out_specs=pl.BlockSpec((BN, DIM), lambda i: (i, 0)),
        out_shape=jax.ShapeDtypeStruct((N, DIM), jnp.float32),
    )(species.reshape(N // BN, 1, BN), wsp_pad)


def _filter_body(de_ref, do_ref, swe_ref, swo_ref,
                 f0w_ref, f0b_ref, f1w_ref, f1b_ref, f2w_ref, f2b_ref,
                 *out_refs):
    de = de_ref[0, 0, :]
    do = do_ref[0, 0, :]
    swe = swe_ref[0, 0, :]
    swo = swo_ref[0, 0, :]
    delta = CUTOFF / NBASIS
    gamma = 1.0 / (2.0 * delta * delta)
    mus = lax.broadcasted_iota(jnp.int32, (EB2, NBASIS), 1).astype(jnp.float32) \
        * (CUTOFF / (NBASIS - 1))
    rb2 = jnp.concatenate(
        [jnp.exp(-gamma * (de[:, None] - mus) ** 2),
         jnp.exp(-gamma * (do[:, None] - mus) ** 2)], axis=1)
    swb = jnp.concatenate(
        [jnp.broadcast_to(swe[:, None], (EB2, DIM)),
         jnp.broadcast_to(swo[:, None], (EB2, DIM))], axis=1)

    def bd(wm, rows, cols):
        z = jnp.zeros((rows, cols), jnp.float32)
        return jnp.concatenate(
            [jnp.concatenate([wm, z], axis=1),
             jnp.concatenate([z, wm], axis=1)], axis=0)

    for l in range(NLAYERS):
        b0 = jnp.concatenate([f0b_ref[l], f0b_ref[l]])[None, :]
        b1 = jnp.concatenate([f1b_ref[l], f1b_ref[l]])[None, :]
        b2 = jnp.concatenate([f2b_ref[l], f2b_ref[l]])[None, :]
        h = _ssp(jnp.dot(rb2, bd(f0w_ref[l], NBASIS, 64),
                         preferred_element_type=jnp.float32) + b0)
        h = _ssp(jnp.dot(h, bd(f1w_ref[l], 64, 64),
                         preferred_element_type=jnp.float32) + b1)
        w = jnp.dot(h, bd(f2w_ref[l], 64, DIM),
                    preferred_element_type=jnp.float32) + b2
        wsp = _ssp(w) * swb
        out_refs[l][...] = wsp


def _filter(distances, switch, f0_W, f0_b, f1_W, f1_b, f2_W, f2_b):
    wspec = lambda s: pl.BlockSpec(s, lambda i: tuple(0 for _ in s))
    oshape = jax.ShapeDtypeStruct((EPAD // 2, 128), jnp.float32)
    d2 = distances.reshape(EPAD // 2, 2)
    s2 = switch.reshape(EPAD // 2, 2)
    espec = pl.BlockSpec((1, 1, EB2), lambda i: (i, 0, 0))
    return pl.pallas_call(
        _filter_body,
        grid=(EPAD // EB,),
        in_specs=[
            espec, espec, espec, espec,
            wspec((NLAYERS, NBASIS, 64)), wspec((NLAYERS, 64)),
            wspec((NLAYERS, 64, 64)), wspec((NLAYERS, 64)),
            wspec((NLAYERS, 64, DIM)), wspec((NLAYERS, DIM)),
        ],
        out_specs=[pl.BlockSpec((EB2, 128), lambda i: (i, 0))] * NLAYERS,
        out_shape=[oshape] * NLAYERS,
    )(d2[:, 0].reshape(EPAD // EB, 1, EB2), d2[:, 1].reshape(EPAD // EB, 1, EB2),
      s2[:, 0].reshape(EPAD // EB, 1, EB2), s2[:, 1].reshape(EPAD // EB, 1, EB2),
      f0_W, f0_b, f1_W, f1_b, f2_W, f2_b)


def _node_in_body(x_ref, w_ref, b_ref, out_ref):
    c = pl.program_id(1)
    y = jnp.dot(x_ref[...], w_ref[...], preferred_element_type=jnp.float32) \
        + b_ref[0, :][None, :]
    out_ref[...] = jnp.where(c == 0, y[:, :HD], y[:, HD:])


def _node_in(xi, w, b):
    return pl.pallas_call(
        _node_in_body,
        grid=(N // BN, NC),
        in_specs=[
            pl.BlockSpec((BN, DIM), lambda i, c: (i, 0)),
            pl.BlockSpec((DIM, DIM), lambda i, c: (0, 0)),
            pl.BlockSpec((1, DIM), lambda i, c: (0, 0)),
        ],
        out_specs=pl.BlockSpec((BN, HD), lambda i, c: (c * (N // BN) + i, 0)),
        out_shape=jax.ShapeDtypeStruct((NC * N, HD), jnp.float32),
    )(xi, w, b.reshape(1, DIM))


def _node_out_body(acc_ref, xp_ref, w2_ref, b2_ref, w3_ref, b3_ref, out_ref):
    t = jnp.concatenate([acc_ref[0, :, :], acc_ref[1, :, :]], axis=-1)
    u = _ssp(jnp.dot(t, w2_ref[...], preferred_element_type=jnp.float32)
             + b2_ref[0, :][None, :])
    out_ref[...] = jnp.dot(u, w3_ref[...], preferred_element_type=jnp.float32) \
        + b3_ref[0, :][None, :] + xp_ref[...]


def _node_out(acc, xi_prev, w2, b2, w3, b3):
    return pl.pallas_call(
        _node_out_body,
        grid=(N // BN,),
        in_specs=[
            pl.BlockSpec((NC, BN, HD), lambda i: (0, i, 0)),
            pl.BlockSpec((BN, DIM), lambda i: (i, 0)),
            pl.BlockSpec((DIM, DIM), lambda i: (0, 0)),
            pl.BlockSpec((1, DIM), lambda i: (0, 0)),
            pl.BlockSpec((DIM, DIM), lambda i: (0, 0)),
            pl.BlockSpec((1, DIM), lambda i: (0, 0)),
        ],
        out_specs=pl.BlockSpec((BN, DIM), lambda i: (i, 0)),
        out_shape=jax.ShapeDtypeStruct((N, DIM), jnp.float32),
    )(acc, xi_prev, w2, b2.reshape(1, DIM), w3, b3.reshape(1, DIM))


def _sc_conv_body(xi2_hbm, wp_hbm, srcr_hbm, dstr_hbm, out_hbm,
                  sidx, didx, g, w, acc, sem):
    c = lax.axis_index("c")
    s = lax.axis_index("s")

    zv = jnp.zeros((16,), jnp.float32)

    def zb_body(r, _):
        g[r, pl.ds(0, 16)] = zv
        g[r, pl.ds(16, 16)] = zv
        return _
    lax.fori_loop(0, ZR, zb_body, 0, unroll=8)
    for q in range(RPT // ZR):
        z0 = pl.multiple_of(s * RPT + q * ZR, ZR)
        pltpu.sync_copy(g.at[pl.ds(0, ZR)], acc.at[pl.ds(z0, ZR)])
    plsc.subcore_barrier()

    def chunk(i, carry):
        base = pl.multiple_of(s * EPT + i * CK, CK)
        rbase = pl.multiple_of(base // G, GR)
        pltpu.sync_copy(srcr_hbm.at[pl.ds(rbase, GR)], sidx)
        pltpu.sync_copy(dstr_hbm.at[c, pl.ds(rbase, GR)], didx)
        cps = [pltpu.async_copy(xi2_hbm.at[didx.at[j]],
                                g.at[pl.ds(j * G, G)], sem)
               for j in range(GR)]
        pltpu.sync_copy(wp_hbm.at[pl.ds(base, CK), pl.ds(c * HD, HD)], w)
        for cp in cps:
            cp.wait()

        def mul(r0, mc):
            for u in range(8):
                r = r0 * 8 + u
                g[r, pl.ds(0, 16)] = g[r, pl.ds(0, 16)] * w[r, pl.ds(0, 16)]
                g[r, pl.ds(16, 16)] = g[r, pl.ds(16, 16)] * w[r, pl.ds(16, 16)]
            return 0
        lax.fori_loop(0, CK // 8, mul, 0)

        for j in range(GR):
            pltpu.sync_copy(g.at[pl.ds(j * G, G)], acc.at[sidx.at[j]], add=True)
        return carry

    lax.fori_loop(0, NCHUNK, chunk, 0)
    plsc.subcore_barrier()
    for q in range(RPT // ZR):
        r0 = pl.multiple_of(s * RPT + q * ZR, ZR)
        pltpu.sync_copy(acc.at[pl.ds(r0, ZR)], out_hbm.at[c, pl.ds(r0, ZR)])


def _sc_conv(xi_h, wp, srcr, dstr):
    mesh = plsc.VectorSubcoreMesh(core_axis_name="c", subcore_axis_name="s")
    return pl.kernel(
        _sc_conv_body,
        out_type=jax.ShapeDtypeStruct((NC, NPAD, HD), jnp.float32),
        mesh=mesh,
        compiler_params=pltpu.CompilerParams(use_tc_tiling_on_sc=False),
        scratch_types=[
            pltpu.VMEM((GR, G), jnp.int32),
            pltpu.VMEM((GR, G), jnp.int32),
            pltpu.VMEM((CK, HD), jnp.float32),
            pltpu.VMEM((CK, HD), jnp.float32),
            pltpu.VMEM_SHARED((NPAD, HD), jnp.float32),
            pltpu.SemaphoreType.DMA,
        ],
    )(xi_h, wp.reshape(EPAD, DIM), srcr, dstr)


def kernel(species, edge_src, edge_dst, distances, switch,
           W_sp, aw1_W, aw1_b, f0_W, f0_b, f1_W, f1_b, f2_W, f2_b,
           aw2_W, aw2_b, aw3_W, aw3_b):
    species = species.astype(jnp.int32)
    edge_src = edge_src.astype(jnp.int32)
    edge_dst = edge_dst.astype(jnp.int32)

    wsp_pad = jnp.zeros((128, DIM), jnp.float32).at[:NSPEC].set(W_sp)
    xi_prev = _embed(species, wsp_pad)

    dpad = jnp.pad(distances, (0, EPAD - E))
    swpad = jnp.pad(switch, (0, EPAD - E))
    wps = _filter(dpad, swpad, f0_W, f0_b, f1_W, f1_b, f2_W, f2_b)

    srcr = jnp.pad(edge_src, (0, EPAD - E)).reshape(EPAD // G, G)
    dstr = jnp.pad(jnp.stack([edge_dst, edge_dst + N]),
                   ((0, 0), (0, EPAD - E))).reshape(NC, EPAD // G, G)

    for l in range(NLAYERS):
        xi_h = _node_in(xi_prev, aw1_W[l], aw1_b[l])
        acc = _sc_conv(xi_h, wps[l], srcr, dstr)
        xi_prev = _node_out(acc, xi_prev, aw2_W[l], aw2_b[l], aw3_W[l], aw3_b[l])
    return xi_prev


# speedup vs baseline: 3.0386x; 1.0761x over previous
import functools

import jax
import jax.numpy as jnp
import numpy as np
from jax import lax
from jax.experimental import pallas as pl
from jax.experimental.pallas import tpu as pltpu
from jax.experimental.pallas import tpu_sc as plsc

N = 50000
E = 800000
DIM = 64
NBASIS = 16
NSPEC = 94
NLAYERS = 3
CUTOFF = 5.0
LOG2 = float(np.log(2.0))

NP2 = 51200
BN = 2048
BP = BN // 2
EB = 4096
EB2 = EB // 2

NC = 2
NS = 16
HD = DIM // NC
CK = 256
G = 128
GR = CK // G
EPAD = 16 * 196 * CK
EPT = EPAD // NS
NCHUNK = EPT // CK
NPAD = 51200
RPT = NPAD // NS
ZR = 200


def _ssp(x):
    t = jnp.minimum(x * np.float32(1.4426950408889634), np.float32(126.0))
    return np.float32(LOG2) * (jnp.log2(1.0 + jnp.exp2(t)) - 1.0)


def _bd2(wm):
    z = jnp.zeros(wm.shape, jnp.float32)
    return jnp.concatenate(
        [jnp.concatenate([wm, z], axis=1),
         jnp.concatenate([z, wm], axis=1)], axis=0)


def _embed_body(sp_ref, wsp_ref, out_ref):
    sp = sp_ref[...]
    ii = lax.broadcasted_iota(jnp.int32, (BP, 128), 1)
    oh = jnp.concatenate(
        [(sp[:, 0][:, None] == ii).astype(jnp.float32),
         (sp[:, 1][:, None] == ii).astype(jnp.float32)], axis=1)
    out_ref[...] = jnp.dot(oh, _bd2(wsp_ref[...]),
                           preferred_element_type=jnp.float32)


def _embed(sp2, wsp_pad):
    return pl.pallas_call(
        _embed_body,
        grid=(NP2 // BN,),
        in_specs=[
            pl.BlockSpec((BP, 2), lambda i: (i, 0)),
            pl.BlockSpec((128, DIM), lambda i: (0, 0)),
        ],
        out_specs=pl.BlockSpec((BP, 128), lambda i: (i, 0)),
        out_shape=jax.ShapeDtypeStruct((NP2 // 2, 128), jnp.float32),
    )(sp2, wsp_pad)


def _filter_body(de_ref, do_ref, swe_ref, swo_ref,
                 f0w_ref, f0b_ref, f1w_ref, f1b_ref, f2w_ref, f2b_ref,
                 *out_refs):
    de = de_ref[0, 0, :]
    do = do_ref[0, 0, :]
    swe = swe_ref[0, 0, :]
    swo = swo_ref[0, 0, :]
    delta = CUTOFF / NBASIS
    gamma = 1.0 / (2.0 * delta * delta)
    mus = lax.broadcasted_iota(jnp.int32, (EB2, NBASIS), 1).astype(jnp.float32) \
        * (CUTOFF / (NBASIS - 1))
    rb2 = jnp.concatenate(
        [jnp.exp(-gamma * (de[:, None] - mus) ** 2),
         jnp.exp(-gamma * (do[:, None] - mus) ** 2)], axis=1)
    swb = jnp.concatenate(
        [jnp.broadcast_to(swe[:, None], (EB2, DIM)),
         jnp.broadcast_to(swo[:, None], (EB2, DIM))], axis=1)

    def bd(wm, rows, cols):
        z = jnp.zeros((rows, cols), jnp.float32)
        return jnp.concatenate(
            [jnp.concatenate([wm, z], axis=1),
             jnp.concatenate([z, wm], axis=1)], axis=0)

    for l in range(NLAYERS):
        b0 = jnp.concatenate([f0b_ref[l], f0b_ref[l]])[None, :]
        b1 = jnp.concatenate([f1b_ref[l], f1b_ref[l]])[None, :]
        b2 = jnp.concatenate([f2b_ref[l], f2b_ref[l]])[None, :]
        h = _ssp(jnp.dot(rb2, bd(f0w_ref[l], NBASIS, 64),
                         preferred_element_type=jnp.float32) + b0)
        h = _ssp(jnp.dot(h, bd(f1w_ref[l], 64, 64),
                         preferred_element_type=jnp.float32) + b1)
        w = jnp.dot(h, bd(f2w_ref[l], 64, DIM),
                    preferred_element_type=jnp.float32) + b2
        wsp = _ssp(w) * swb
        out_refs[l][...] = wsp


def _filter(distances, switch, f0_W, f0_b, f1_W, f1_b, f2_W, f2_b):
    wspec = lambda s: pl.BlockSpec(s, lambda i: tuple(0 for _ in s))
    oshape = jax.ShapeDtypeStruct((EPAD // 2, 128), jnp.float32)
    d2 = distances.reshape(EPAD // 2, 2)
    s2 = switch.reshape(EPAD // 2, 2)
    espec = pl.BlockSpec((1, 1, EB2), lambda i: (i, 0, 0))
    return pl.pallas_call(
        _filter_body,
        grid=(EPAD // EB,),
        in_specs=[
            espec, espec, espec, espec,
            wspec((NLAYERS, NBASIS, 64)), wspec((NLAYERS, 64)),
            wspec((NLAYERS, 64, 64)), wspec((NLAYERS, 64)),
            wspec((NLAYERS, 64, DIM)), wspec((NLAYERS, DIM)),
        ],
        out_specs=[pl.BlockSpec((EB2, 128), lambda i: (i, 0))] * NLAYERS,
        out_shape=[oshape] * NLAYERS,
    )(d2[:, 0].reshape(EPAD // EB, 1, EB2), d2[:, 1].reshape(EPAD // EB, 1, EB2),
      s2[:, 0].reshape(EPAD // EB, 1, EB2), s2[:, 1].reshape(EPAD // EB, 1, EB2),
      f0_W, f0_b, f1_W, f1_b, f2_W, f2_b)


def _node_in_body(x_ref, w_ref, b_ref, out_ref):
    bt = jnp.concatenate([b_ref[0, :], b_ref[0, :]])[None, :]
    out_ref[...] = jnp.dot(x_ref[...], _bd2(w_ref[...]),
                           preferred_element_type=jnp.float32) + bt


def _node_in(x2p, w, b):
    return pl.pallas_call(
        _node_in_body,
        grid=(NP2 // BN,),
        in_specs=[
            pl.BlockSpec((BP, 128), lambda i: (i, 0)),
            pl.BlockSpec((DIM, DIM), lambda i: (0, 0)),
            pl.BlockSpec((1, DIM), lambda i: (0, 0)),
        ],
        out_specs=pl.BlockSpec((BP, 128), lambda i: (i, 0)),
        out_shape=jax.ShapeDtypeStruct((NP2 // 2, 128), jnp.float32),
    )(x2p, w, b.reshape(1, DIM))


def _node_out_body(acc_ref, xp_ref, w2_ref, b2_ref, w3_ref, b3_ref, out_ref):
    b2t = jnp.concatenate([b2_ref[0, :], b2_ref[0, :]])[None, :]
    b3t = jnp.concatenate([b3_ref[0, :], b3_ref[0, :]])[None, :]
    u = _ssp(jnp.dot(acc_ref[...], _bd2(w2_ref[...]),
                     preferred_element_type=jnp.float32) + b2t)
    out_ref[...] = jnp.dot(u, _bd2(w3_ref[...]),
                           preferred_element_type=jnp.float32) + b3t + xp_ref[...]


def _node_out(acc2p, xp2p, w2, b2, w3, b3):
    return pl.pallas_call(
        _node_out_body,
        grid=(NP2 // BN,),
        in_specs=[
            pl.BlockSpec((BP, 128), lambda i: (i, 0)),
            pl.BlockSpec((BP, 128), lambda i: (i, 0)),
            pl.BlockSpec((DIM, DIM), lambda i: (0, 0)),
            pl.BlockSpec((1, DIM), lambda i: (0, 0)),
            pl.BlockSpec((DIM, DIM), lambda i: (0, 0)),
            pl.BlockSpec((1, DIM), lambda i: (0, 0)),
        ],
        out_specs=pl.BlockSpec((BP, 128), lambda i: (i, 0)),
        out_shape=jax.ShapeDtypeStruct((NP2 // 2, 128), jnp.float32),
    )(acc2p, xp2p, w2, b2.reshape(1, DIM), w3, b3.reshape(1, DIM))


def _sc_conv_body(xi2_hbm, wp_hbm, srcr_hbm, dstr_hbm, out_hbm,
                  sidx, didx, g, w, acc, sem):
    c = lax.axis_index("c")
    s = lax.axis_index("s")

    zv = jnp.zeros((16,), jnp.float32)

    def zb_body(r, _):
        g[r, pl.ds(0, 16)] = zv
        g[r, pl.ds(16, 16)] = zv
        return _
    lax.fori_loop(0, ZR, zb_body, 0, unroll=8)
    for q in range(RPT // ZR):
        z0 = pl.multiple_of(s * RPT + q * ZR, ZR)
        pltpu.sync_copy(g.at[pl.ds(0, ZR)], acc.at[pl.ds(z0, ZR)])
    plsc.subcore_barrier()

    def chunk(i, carry):
        base = pl.multiple_of(s * EPT + i * CK, CK)
        rbase = pl.multiple_of(base // G, GR)
        pltpu.sync_copy(srcr_hbm.at[pl.ds(rbase, GR)], sidx)
        pltpu.sync_copy(dstr_hbm.at[c, pl.ds(rbase, GR)], didx)
        cps = [pltpu.async_copy(xi2_hbm.at[didx.at[j]],
                                g.at[pl.ds(j * G, G)], sem)
               for j in range(GR)]
        pltpu.sync_copy(wp_hbm.at[pl.ds(base, CK), pl.ds(c * HD, HD)], w)
        for cp in cps:
            cp.wait()

        def mul(r0, mc):
            for u in range(8):
                r = r0 * 8 + u
                g[r, pl.ds(0, 16)] = g[r, pl.ds(0, 16)] * w[r, pl.ds(0, 16)]
                g[r, pl.ds(16, 16)] = g[r, pl.ds(16, 16)] * w[r, pl.ds(16, 16)]
            return 0
        lax.fori_loop(0, CK // 8, mul, 0)

        for j in range(GR):
            pltpu.sync_copy(g.at[pl.ds(j * G, G)], acc.at[sidx.at[j]], add=True)
        return carry

    lax.fori_loop(0, NCHUNK, chunk, 0)
    plsc.subcore_barrier()
    for q in range(RPT // ZR):
        r0 = pl.multiple_of(s * RPT + q * ZR, ZR)
        pltpu.sync_copy(acc.at[pl.ds(r0, ZR)], out_hbm.at[pl.ds(r0, ZR), c])


def _sc_conv(xi_h, wp, srcr, dstr):
    mesh = plsc.VectorSubcoreMesh(core_axis_name="c", subcore_axis_name="s")
    return pl.kernel(
        _sc_conv_body,
        out_type=jax.ShapeDtypeStruct((NPAD, NC, HD), jnp.float32),
        mesh=mesh,
        compiler_params=pltpu.CompilerParams(use_tc_tiling_on_sc=False),
        scratch_types=[
            pltpu.VMEM((GR, G), jnp.int32),
            pltpu.VMEM((GR, G), jnp.int32),
            pltpu.VMEM((CK, HD), jnp.float32),
            pltpu.VMEM((CK, HD), jnp.float32),
            pltpu.VMEM_SHARED((NPAD, HD), jnp.float32),
            pltpu.SemaphoreType.DMA,
        ],
    )(xi_h.reshape(2 * NP2, HD), wp.reshape(EPAD, DIM), srcr, dstr)


def kernel(species, edge_src, edge_dst, distances, switch,
           W_sp, aw1_W, aw1_b, f0_W, f0_b, f1_W, f1_b, f2_W, f2_b,
           aw2_W, aw2_b, aw3_W, aw3_b):
    species = species.astype(jnp.int32)
    edge_src = edge_src.astype(jnp.int32)
    edge_dst = edge_dst.astype(jnp.int32)

    wsp_pad = jnp.zeros((128, DIM), jnp.float32).at[:NSPEC].set(W_sp)
    sp2 = jnp.pad(species, (0, NP2 - N)).reshape(NP2 // 2, 2)
    xi2p = _embed(sp2, wsp_pad)

    dpad = jnp.pad(distances, (0, EPAD - E))
    swpad = jnp.pad(switch, (0, EPAD - E))
    wps = _filter(dpad, swpad, f0_W, f0_b, f1_W, f1_b, f2_W, f2_b)

    srcr = jnp.pad(edge_src, (0, EPAD - E)).reshape(EPAD // G, G)
    dstr = jnp.pad(jnp.stack([2 * edge_dst, 2 * edge_dst + 1]),
                   ((0, 0), (0, EPAD - E))).reshape(NC, EPAD // G, G)

    for l in range(NLAYERS):
        y2p = _node_in(xi2p, aw1_W[l], aw1_b[l])
        acc = _sc_conv(y2p, wps[l], srcr, dstr)
        xi2p = _node_out(acc.reshape(NP2 // 2, 128), xi2p,
                         aw2_W[l], aw2_b[l], aw3_W[l], aw3_b[l])
    return xi2p.reshape(NP2, DIM)[:N]


# speedup vs baseline: 3.3730x; 1.1101x over previous
import functools

import jax
import jax.numpy as jnp
import numpy as np
from jax import lax
from jax.experimental import pallas as pl
from jax.experimental.pallas import tpu as pltpu
from jax.experimental.pallas import tpu_sc as plsc

N = 50000
E = 800000
DIM = 64
NBASIS = 16
NSPEC = 94
NLAYERS = 3
CUTOFF = 5.0
LOG2 = float(np.log(2.0))

NP2 = 51200
BN = 2048
BP = BN // 2
EB = 3072
EB2 = EB // 2

NC = 2
NS = 16
HD = DIM // NC
CK = 192
G = 96
GR = CK // G
SCH = 8
NSUP = 33
NCHUNK = SCH * NSUP
EPAD = 16 * NCHUNK * CK
EPT = EPAD // NS
NPAD = 51200
RPT = NPAD // NS
ZR = 160


def _ssp(x):
    t = jnp.minimum(x * np.float32(1.4426950408889634), np.float32(126.0))
    return np.float32(LOG2) * (jnp.log2(1.0 + jnp.exp2(t)) - 1.0)


def _bd2(wm):
    z = jnp.zeros(wm.shape, jnp.float32)
    return jnp.concatenate(
        [jnp.concatenate([wm, z], axis=1),
         jnp.concatenate([z, wm], axis=1)], axis=0)


def _embed_body(sp_ref, wsp_ref, out_ref):
    sp = sp_ref[...]
    ii = lax.broadcasted_iota(jnp.int32, (BP, 128), 1)
    oh = jnp.concatenate(
        [(sp[:, 0][:, None] == ii).astype(jnp.float32),
         (sp[:, 1][:, None] == ii).astype(jnp.float32)], axis=1)
    out_ref[...] = jnp.dot(oh, _bd2(wsp_ref[...]),
                           preferred_element_type=jnp.float32)


def _embed(sp2, wsp_pad):
    return pl.pallas_call(
        _embed_body,
        grid=(NP2 // BN,),
        in_specs=[
            pl.BlockSpec((BP, 2), lambda i: (i, 0)),
            pl.BlockSpec((128, DIM), lambda i: (0, 0)),
        ],
        out_specs=pl.BlockSpec((BP, 128), lambda i: (i, 0)),
        out_shape=jax.ShapeDtypeStruct((NP2 // 2, 128), jnp.float32),
    )(sp2, wsp_pad)


def _filter_body(de_ref, do_ref, swe_ref, swo_ref,
                 f0w_ref, f0b_ref, f1w_ref, f1b_ref, f2w_ref, f2b_ref,
                 *out_refs):
    de = de_ref[0, 0, :]
    do = do_ref[0, 0, :]
    swe = swe_ref[0, 0, :]
    swo = swo_ref[0, 0, :]
    delta = CUTOFF / NBASIS
    gamma = 1.0 / (2.0 * delta * delta)
    mus = lax.broadcasted_iota(jnp.int32, (EB2, NBASIS), 1).astype(jnp.float32) \
        * (CUTOFF / (NBASIS - 1))
    rb2 = jnp.concatenate(
        [jnp.exp(-gamma * (de[:, None] - mus) ** 2),
         jnp.exp(-gamma * (do[:, None] - mus) ** 2)], axis=1)
    swb = jnp.concatenate(
        [jnp.broadcast_to(swe[:, None], (EB2, DIM)),
         jnp.broadcast_to(swo[:, None], (EB2, DIM))], axis=1)

    def bd(wm, rows, cols):
        z = jnp.zeros((rows, cols), jnp.float32)
        return jnp.concatenate(
            [jnp.concatenate([wm, z], axis=1),
             jnp.concatenate([z, wm], axis=1)], axis=0)

    for l in range(NLAYERS):
        b0 = jnp.concatenate([f0b_ref[l], f0b_ref[l]])[None, :]
        b1 = jnp.concatenate([f1b_ref[l], f1b_ref[l]])[None, :]
        b2 = jnp.concatenate([f2b_ref[l], f2b_ref[l]])[None, :]
        h = _ssp(jnp.dot(rb2, bd(f0w_ref[l], NBASIS, 64),
                         preferred_element_type=jnp.float32) + b0)
        h = _ssp(jnp.dot(h, bd(f1w_ref[l], 64, 64),
                         preferred_element_type=jnp.float32) + b1)
        w = jnp.dot(h, bd(f2w_ref[l], 64, DIM),
                    preferred_element_type=jnp.float32) + b2
        wsp = _ssp(w) * swb
        out_refs[l][...] = wsp


def _filter(distances, switch, f0_W, f0_b, f1_W, f1_b, f2_W, f2_b):
    wspec = lambda s: pl.BlockSpec(s, lambda i: tuple(0 for _ in s))
    oshape = jax.ShapeDtypeStruct((EPAD // 2, 128), jnp.float32)
    d2 = distances.reshape(EPAD // 2, 2)
    s2 = switch.reshape(EPAD // 2, 2)
    espec = pl.BlockSpec((1, 1, EB2), lambda i: (i, 0, 0))
    return pl.pallas_call(
        _filter_body,
        grid=(EPAD // EB,),
        in_specs=[
            espec, espec, espec, espec,
            wspec((NLAYERS, NBASIS, 64)), wspec((NLAYERS, 64)),
            wspec((NLAYERS, 64, 64)), wspec((NLAYERS, 64)),
            wspec((NLAYERS, 64, DIM)), wspec((NLAYERS, DIM)),
        ],
        out_specs=[pl.BlockSpec((EB2, 128), lambda i: (i, 0))] * NLAYERS,
        out_shape=[oshape] * NLAYERS,
    )(d2[:, 0].reshape(EPAD // EB, 1, EB2), d2[:, 1].reshape(EPAD // EB, 1, EB2),
      s2[:, 0].reshape(EPAD // EB, 1, EB2), s2[:, 1].reshape(EPAD // EB, 1, EB2),
      f0_W, f0_b, f1_W, f1_b, f2_W, f2_b)


def _node_in_body(x_ref, w_ref, b_ref, out_ref):
    bt = jnp.concatenate([b_ref[0, :], b_ref[0, :]])[None, :]
    out_ref[...] = jnp.dot(x_ref[...], _bd2(w_ref[...]),
                           preferred_element_type=jnp.float32) + bt


def _node_in(x2p, w, b):
    return pl.pallas_call(
        _node_in_body,
        grid=(NP2 // BN,),
        in_specs=[
            pl.BlockSpec((BP, 128), lambda i: (i, 0)),
            pl.BlockSpec((DIM, DIM), lambda i: (0, 0)),
            pl.BlockSpec((1, DIM), lambda i: (0, 0)),
        ],
        out_specs=pl.BlockSpec((BP, 128), lambda i: (i, 0)),
        out_shape=jax.ShapeDtypeStruct((NP2 // 2, 128), jnp.float32),
    )(x2p, w, b.reshape(1, DIM))


def _node_out_body(acc_ref, xp_ref, w2_ref, b2_ref, w3_ref, b3_ref, out_ref):
    b2t = jnp.concatenate([b2_ref[0, :], b2_ref[0, :]])[None, :]
    b3t = jnp.concatenate([b3_ref[0, :], b3_ref[0, :]])[None, :]
    u = _ssp(jnp.dot(acc_ref[...], _bd2(w2_ref[...]),
                     preferred_element_type=jnp.float32) + b2t)
    out_ref[...] = jnp.dot(u, _bd2(w3_ref[...]),
                           preferred_element_type=jnp.float32) + b3t + xp_ref[...]


def _node_out(acc2p, xp2p, w2, b2, w3, b3):
    return pl.pallas_call(
        _node_out_body,
        grid=(NP2 // BN,),
        in_specs=[
            pl.BlockSpec((BP, 128), lambda i: (i, 0)),
            pl.BlockSpec((BP, 128), lambda i: (i, 0)),
            pl.BlockSpec((DIM, DIM), lambda i: (0, 0)),
            pl.BlockSpec((1, DIM), lambda i: (0, 0)),
            pl.BlockSpec((DIM, DIM), lambda i: (0, 0)),
            pl.BlockSpec((1, DIM), lambda i: (0, 0)),
        ],
        out_specs=pl.BlockSpec((BP, 128), lambda i: (i, 0)),
        out_shape=jax.ShapeDtypeStruct((NP2 // 2, 128), jnp.float32),
    )(acc2p, xp2p, w2, b2.reshape(1, DIM), w3, b3.reshape(1, DIM))


def _sc_conv_body(xi2_hbm, wp_hbm, srcr_hbm, dstr_hbm, out_hbm,
                  sidx, didx, g0, g1, w0, w1, acc, sem0, sem1):
    c = lax.axis_index("c")
    s = lax.axis_index("s")
    gb = (g0, g1)
    wb = (w0, w1)
    sems = (sem0, sem1)

    zv = jnp.zeros((16,), jnp.float32)

    def zb_body(r, zc):
        g0[r, pl.ds(0, 16)] = zv
        g0[r, pl.ds(16, 16)] = zv
        return zc
    lax.fori_loop(0, ZR, zb_body, 0, unroll=8)
    for q in range(RPT // ZR):
        z0 = pl.multiple_of(s * RPT + q * ZR, ZR)
        pltpu.sync_copy(g0.at[pl.ds(0, ZR)], acc.at[pl.ds(z0, ZR)])
    plsc.subcore_barrier()

    tbase = s * EPT

    def load_idx(k):
        rb = pl.multiple_of((tbase + k * SCH * CK) // G, 2 * SCH)
        pltpu.sync_copy(srcr_hbm.at[pl.ds(rb, 2 * SCH)], sidx)
        pltpu.sync_copy(dstr_hbm.at[c, pl.ds(rb, 2 * SCH)], didx)

    def issue(k, p, par):
        base = pl.multiple_of(tbase + (k * SCH + p) * CK, CK)
        for j in range(GR):
            pltpu.async_copy(xi2_hbm.at[didx.at[GR * p + j]],
                             gb[par].at[pl.ds(j * G, G)], sems[par])
        pltpu.async_copy(wp_hbm.at[pl.ds(base, CK), pl.ds(c * HD, HD)],
                         wb[par], sems[par])

    def wait(k, p, par):
        for j in range(GR):
            pltpu.make_async_copy(xi2_hbm.at[didx.at[GR * p + j]],
                                  gb[par].at[pl.ds(j * G, G)], sems[par]).wait()
        base = pl.multiple_of(tbase + (k * SCH + p) * CK, CK)
        pltpu.make_async_copy(wp_hbm.at[pl.ds(base, CK), pl.ds(c * HD, HD)],
                              wb[par], sems[par]).wait()

    load_idx(0)
    issue(0, 0, 0)

    def sup(k, carry):
        for p in range(SCH):
            par = p & 1
            if p < SCH - 1:
                issue(k, p + 1, 1 - par)
            wait(k, p, par)
            g, w = gb[par], wb[par]

            def mul(r0, mc):
                for u in range(8):
                    r = r0 * 8 + u
                    g[r, pl.ds(0, 16)] = g[r, pl.ds(0, 16)] * w[r, pl.ds(0, 16)]
                    g[r, pl.ds(16, 16)] = g[r, pl.ds(16, 16)] * w[r, pl.ds(16, 16)]
                return mc
            lax.fori_loop(0, CK // 8, mul, 0)

            for j in range(GR):
                pltpu.sync_copy(g.at[pl.ds(j * G, G)],
                                acc.at[sidx.at[GR * p + j]], add=True)
            if p == SCH - 1:
                @pl.when(k + 1 < NSUP)
                def _next():
                    load_idx(k + 1)
                    issue(k + 1, 0, 1 - par)
        return carry

    lax.fori_loop(0, NSUP, sup, 0)
    plsc.subcore_barrier()
    for q in range(RPT // ZR):
        r0 = pl.multiple_of(s * RPT + q * ZR, ZR)
        pltpu.sync_copy(acc.at[pl.ds(r0, ZR)], out_hbm.at[pl.ds(r0, ZR), c])


def _sc_conv(xi_h, wp, srcr, dstr):
    mesh = plsc.VectorSubcoreMesh(core_axis_name="c", subcore_axis_name="s")
    return pl.kernel(
        _sc_conv_body,
        out_type=jax.ShapeDtypeStruct((NPAD, NC, HD), jnp.float32),
        mesh=mesh,
        compiler_params=pltpu.CompilerParams(use_tc_tiling_on_sc=False),
        scratch_types=[
            pltpu.VMEM((GR * SCH, G), jnp.int32),
            pltpu.VMEM((GR * SCH, G), jnp.int32),
            pltpu.VMEM((CK, HD), jnp.float32),
            pltpu.VMEM((CK, HD), jnp.float32),
            pltpu.VMEM((CK, HD), jnp.float32),
            pltpu.VMEM((CK, HD), jnp.float32),
            pltpu.VMEM_SHARED((NPAD, HD), jnp.float32),
            pltpu.SemaphoreType.DMA,
            pltpu.SemaphoreType.DMA,
        ],
    )(xi_h.reshape(2 * NP2, HD), wp.reshape(EPAD, DIM), srcr, dstr)


def kernel(species, edge_src, edge_dst, distances, switch,
           W_sp, aw1_W, aw1_b, f0_W, f0_b, f1_W, f1_b, f2_W, f2_b,
           aw2_W, aw2_b, aw3_W, aw3_b):
    species = species.astype(jnp.int32)
    edge_src = edge_src.astype(jnp.int32)
    edge_dst = edge_dst.astype(jnp.int32)

    wsp_pad = jnp.zeros((128, DIM), jnp.float32).at[:NSPEC].set(W_sp)
    sp2 = jnp.pad(species, (0, NP2 - N)).reshape(NP2 // 2, 2)
    xi2p = _embed(sp2, wsp_pad)

    dpad = jnp.pad(distances, (0, EPAD - E))
    swpad = jnp.pad(switch, (0, EPAD - E))
    wps = _filter(dpad, swpad, f0_W, f0_b, f1_W, f1_b, f2_W, f2_b)

    srcr = jnp.pad(edge_src, (0, EPAD - E)).reshape(EPAD // G, G)
    dstr = jnp.pad(jnp.stack([2 * edge_dst, 2 * edge_dst + 1]),
                   ((0, 0), (0, EPAD - E))).reshape(NC, EPAD // G, G)

    for l in range(NLAYERS):
        y2p = _node_in(xi2p, aw1_W[l], aw1_b[l])
        acc = _sc_conv(y2p, wps[l], srcr, dstr)
        xi2p = _node_out(acc.reshape(NP2 // 2, 128), xi2p,
                         aw2_W[l], aw2_b[l], aw3_W[l], aw3_b[l])
    return xi2p.reshape(NP2, DIM)[:N]
